# Initial kernel scaffold; baseline (speedup 1.0000x reference)
import jax
import jax.numpy as jnp
from jax.experimental import pallas as pl


def kernel(x, edge_index, W1, b1, g1, bt1, W2, b2, g2, bt2, W3, b3):
    raise NotImplementedError("write your pallas kernel here")


# speedup vs baseline: 12.1001x; 12.1001x over previous
import functools
import jax
import jax.numpy as jnp
from jax import lax
from jax.experimental import pallas as pl
from jax.experimental.pallas import tpu as pltpu
from jax.experimental.pallas import tpu_sc as plsc

NN = 10000
DD = 128
EE = 320000
NC = 2
NS = 16
NW = NC * NS
EPW = EE // NW
BB = 128
NB = -(-EPW // BB)
STRIPE = 632
ACC_ROWS = NS * STRIPE
TRASH = NN
DW = 16


def _mesh():
    return plsc.VectorSubcoreMesh(core_axis_name="c", subcore_axis_name="s")


@functools.partial(
    pl.kernel,
    out_type=jax.ShapeDtypeStruct((NC, ACC_ROWS, DW), jnp.float32),
    mesh=_mesh(),
    scratch_types=[
        pltpu.VMEM((NB, BB), jnp.int32),
        pltpu.VMEM((BB, DW), jnp.float32),
        pltpu.VMEM_SHARED((ACC_ROWS, DW), jnp.float32),
    ],
)
def _sc_degree(dst_hbm, out_hbm, dst_v, ones_v, acc):
    c = lax.axis_index("c")
    s = lax.axis_index("s")
    w = c * NS + s
    pltpu.sync_copy(dst_hbm.at[w], dst_v)

    def fill_zero(i, carry):
        ones_v[i, pl.ds(0, 16)] = jnp.zeros((16,), jnp.float32)
        return carry

    lax.fori_loop(0, BB, fill_zero, 0)
    base = s * STRIPE
    for off, sz in ((0, 128), (128, 128), (256, 128), (384, 128), (512, 120)):
        pltpu.sync_copy(ones_v.at[pl.ds(0, sz)], acc.at[pl.ds(base + off, sz)])

    def fill_one(i, carry):
        ones_v[i, pl.ds(0, 16)] = jnp.ones((16,), jnp.float32)
        return carry

    lax.fori_loop(0, BB, fill_one, 0)
    plsc.subcore_barrier()

    def body(j, carry):
        pltpu.sync_copy(ones_v, acc.at[dst_v.at[j]], add=True)
        return carry

    lax.fori_loop(0, NB, body, 0)
    plsc.subcore_barrier()
    pltpu.sync_copy(acc.at[pl.ds(base, STRIPE)], out_hbm.at[c, pl.ds(base, STRIPE)])


@functools.partial(
    pl.kernel,
    out_type=jax.ShapeDtypeStruct((NC, ACC_ROWS, DD), jnp.float32),
    mesh=_mesh(),
    scratch_types=[
        pltpu.VMEM((NB, BB), jnp.int32),
        pltpu.VMEM((NB, BB), jnp.int32),
        pltpu.VMEM((BB, DD), jnp.float32),
        pltpu.VMEM_SHARED((ACC_ROWS, DD), jnp.float32),
        pltpu.SemaphoreType.DMA,
    ],
)
def _sc_spmm(h_hbm, src_hbm, dst_hbm, out_hbm, src_v, dst_v, rows_v, acc, sem):
    c = lax.axis_index("c")
    s = lax.axis_index("s")
    w = c * NS + s
    pltpu.sync_copy(src_hbm.at[w], src_v)
    pltpu.sync_copy(dst_hbm.at[w], dst_v)

    def fill_zero(i, carry):
        for k in range(DD // 16):
            rows_v[i, pl.ds(k * 16, 16)] = jnp.zeros((16,), jnp.float32)
        return carry

    lax.fori_loop(0, BB, fill_zero, 0)
    base = s * STRIPE
    for off, sz in ((0, 128), (128, 128), (256, 128), (384, 128), (512, 120)):
        pltpu.sync_copy(rows_v.at[pl.ds(0, sz)], acc.at[pl.ds(base + off, sz)])
    plsc.subcore_barrier()

    def body(j, carry):
        pltpu.async_copy(h_hbm.at[src_v.at[j]], rows_v, sem).wait()
        pltpu.sync_copy(rows_v, acc.at[dst_v.at[j]], add=True)
        return carry

    lax.fori_loop(0, NB, body, 0)
    plsc.subcore_barrier()
    pltpu.sync_copy(acc.at[pl.ds(base, STRIPE)], out_hbm.at[c, pl.ds(base, STRIPE)])


def _tc_matmul_body(x_ref, w_ref, o_ref):
    o_ref[...] = jnp.dot(x_ref[...], w_ref[...],
                         preferred_element_type=jnp.float32)


def _dinv_from(degp):
    deg = degp[0, :NN, 0:1] + degp[1, :NN, 0:1] + 1.0
    return lax.rsqrt(deg)


def _tc_scale_body(degp_ref, y_ref, o_ref):
    o_ref[...] = y_ref[...] * _dinv_from(degp_ref[...])


def _tc_mid_body(degp_ref, p_ref, h_ref, b_ref, g_ref, bt_ref, w_ref, o_ref):
    dinv = _dinv_from(degp_ref[...])
    p = p_ref[...]
    agg = dinv * (p[0, :NN] + p[1, :NN] + h_ref[...]) + b_ref[...]
    mu = jnp.mean(agg, axis=0, keepdims=True)
    cen = agg - mu
    var = jnp.mean(cen * cen, axis=0, keepdims=True)
    z = g_ref[...] * cen * lax.rsqrt(var + 1e-5) + bt_ref[...]
    r = jnp.maximum(z, 0.0)
    o_ref[...] = jnp.dot(r, w_ref[...],
                         preferred_element_type=jnp.float32) * dinv


def _tc_final_body(degp_ref, p_ref, h_ref, b_ref, o_ref):
    dinv = _dinv_from(degp_ref[...])
    p = p_ref[...]
    o_ref[...] = dinv * (p[0, :NN] + p[1, :NN] + h_ref[...]) + b_ref[...]


def _f32(shape):
    return jax.ShapeDtypeStruct(shape, jnp.float32)


_tc_matmul = pl.pallas_call(_tc_matmul_body, out_shape=_f32((NN, DD)))
_tc_scale = pl.pallas_call(_tc_scale_body, out_shape=_f32((NN, DD)))
_tc_mid = pl.pallas_call(_tc_mid_body, out_shape=_f32((NN, DD)))
_tc_final = pl.pallas_call(_tc_final_body, out_shape=_f32((NN, DD)))


def kernel(x, edge_index, W1, b1, g1, bt1, W2, b2, g2, bt2, W3, b3):
    src = edge_index[0].reshape(NW, EPW)
    dst = edge_index[1].reshape(NW, EPW)
    pad = NB * BB - EPW
    src = jnp.pad(src, ((0, 0), (0, pad))).reshape(NW, NB, BB)
    dst = jnp.pad(dst, ((0, 0), (0, pad)),
                  constant_values=TRASH).reshape(NW, NB, BB)

    degp = _sc_degree(dst)
    y1 = _tc_matmul(x, W1)
    h1 = _tc_scale(degp, y1)
    p1 = _sc_spmm(h1, src, dst)
    h2 = _tc_mid(degp, p1, h1, b1.reshape(1, DD), g1.reshape(1, DD),
                 bt1.reshape(1, DD), W2)
    p2 = _sc_spmm(h2, src, dst)
    h3 = _tc_mid(degp, p2, h2, b2.reshape(1, DD), g2.reshape(1, DD),
                 bt2.reshape(1, DD), W3)
    p3 = _sc_spmm(h3, src, dst)
    return _tc_final(degp, p3, h3, b3.reshape(1, DD))
